# baseline (speedup 1.0000x reference)
import functools

import jax
import jax.numpy as jnp
from jax import lax
from jax.experimental import pallas as pl
from jax.experimental.pallas import tpu as pltpu
from jax.experimental.pallas import tpu_sc as plsc

N = 10000
D = 128
H = D // 2
E = 320000

NPAD = 10240
EPAD = 327680
CHUNKS = 80
CHUNKS2 = 40
ROWS_PER_TILE = NPAD // 16
BR = 512
GRID = NPAD // BR

_mesh = plsc.VectorSubcoreMesh(core_axis_name="c", subcore_axis_name="s")
_f32 = jnp.float32
_sc_params = pltpu.CompilerParams(use_tc_tiling_on_sc=False)


@functools.partial(
    pl.kernel,
    out_type=jax.ShapeDtypeStruct((2, NPAD, 8), _f32),
    mesh=_mesh,
    scratch_types=[
        pltpu.VMEM((CHUNKS, 128), jnp.int32),
        pltpu.VMEM((128, 8), _f32),
        pltpu.VMEM((ROWS_PER_TILE, 8), _f32),
        pltpu.VMEM_SHARED((NPAD, 8), _f32),
    ],
    compiler_params=_sc_params,
)
def _hist(rows_hbm, ones8_hbm, zeros8_hbm, out_hbm, rowbuf, onesv, stage, acc):
    cid = lax.axis_index("c")
    sid = lax.axis_index("s")
    w = sid * 2 + cid
    pltpu.sync_copy(rows_hbm.at[pl.ds(w * CHUNKS, CHUNKS)], rowbuf)
    pltpu.sync_copy(ones8_hbm, onesv)
    pltpu.sync_copy(zeros8_hbm, stage)
    pltpu.sync_copy(stage, acc.at[pl.ds(sid * ROWS_PER_TILE, ROWS_PER_TILE)])
    plsc.subcore_barrier()

    def body(c, carry):
        pltpu.sync_copy(onesv, acc.at[rowbuf.at[c]], add=True)
        return carry

    lax.fori_loop(0, CHUNKS, body, 0)
    plsc.subcore_barrier()
    pltpu.sync_copy(acc.at[pl.ds(sid * ROWS_PER_TILE, ROWS_PER_TILE)], stage)
    pltpu.sync_copy(stage, out_hbm.at[cid, pl.ds(sid * ROWS_PER_TILE, ROWS_PER_TILE)])


@functools.partial(
    pl.kernel,
    out_type=jax.ShapeDtypeStruct((2, NPAD, D), _f32),
    mesh=_mesh,
    scratch_types=[
        pltpu.VMEM((CHUNKS2, 256), jnp.int32),
        pltpu.VMEM((CHUNKS2, 256), jnp.int32),
        pltpu.VMEM((256, H), _f32),
        pltpu.VMEM((256, H), _f32),
        pltpu.VMEM((256, H), _f32),
        pltpu.VMEM((256, H), _f32),
        pltpu.VMEM_SHARED((NPAD, H), _f32),
        pltpu.SemaphoreType.DMA,
        pltpu.SemaphoreType.DMA,
        pltpu.SemaphoreType.DMA,
        pltpu.SemaphoreType.DMA,
        pltpu.SemaphoreType.DMA,
        pltpu.SemaphoreType.DMA,
        pltpu.SemaphoreType.DMA,
        pltpu.SemaphoreType.DMA,
    ],
    compiler_params=_sc_params,
)
def _gs(g2_hbm, rowsa_hbm, rowsb_hbm, cols_hbm, zeros_hbm, out_hbm,
        rowbuf, colbuf, gb0, gb1, gb2, gb3, acc,
        gs0, gs1, gs2, gs3, ss0, ss1, ss2, ss3):
    cid = lax.axis_index("c")
    sid = lax.axis_index("s")
    w = sid * 2 + cid
    gbufs = (gb0, gb1, gb2, gb3)
    gsems = (gs0, gs1, gs2, gs3)
    ssems = (ss0, ss1, ss2, ss3)
    pltpu.sync_copy(cols_hbm.at[pl.ds(w * CHUNKS2, CHUNKS2)], colbuf)

    for rows_hbm, coff in ((rowsa_hbm, 0), (rowsb_hbm, H)):
        pltpu.sync_copy(rows_hbm.at[pl.ds(w * CHUNKS2, CHUNKS2)], rowbuf)
        pltpu.sync_copy(zeros_hbm, acc.at[pl.ds(sid * ROWS_PER_TILE, ROWS_PER_TILE)])
        plsc.subcore_barrier()

        def _gather_desc(c, b):
            return pltpu.make_async_copy(g2_hbm.at[rowbuf.at[c]], gbufs[b], gsems[b])

        def _scatter_desc(c, b):
            return pltpu.make_async_copy(gbufs[b], acc.at[colbuf.at[c]], ssems[b])

        def _step(c, b):
            _gather_desc(c, b).wait()
            _scatter_desc(c, b).start(add=True)
            b2 = (b + 3) % 4

            @pl.when(c >= 1)
            def _():
                _scatter_desc(c - 1, b2).wait()

            @pl.when(c + 3 < CHUNKS2)
            def _():
                _gather_desc(c + 3, b2).start()

        for b0 in range(3):
            _gather_desc(b0, b0).start()

        def body(g, carry):
            for b in range(4):
                _step(g * 4 + b, b)
            return carry

        lax.fori_loop(0, CHUNKS2 // 4, body, 0)
        _scatter_desc(CHUNKS2 - 1, (CHUNKS2 - 1) % 4).wait()
        plsc.subcore_barrier()
        pltpu.sync_copy(
            acc.at[pl.ds(sid * ROWS_PER_TILE, ROWS_PER_TILE)],
            out_hbm.at[cid, pl.ds(sid * ROWS_PER_TILE, ROWS_PER_TILE), pl.ds(coff, H)])


def _dinv_block(deg_ref, i):
    deg = deg_ref[0] + deg_ref[1] + 1.0
    rows = lax.broadcasted_iota(jnp.int32, (BR, 1), 0) + i * BR
    return jnp.where(rows < N, lax.rsqrt(deg), 0.0)


def _dense1_body(deg_ref, x_ref, w_ref, b_ref, g_ref):
    dinv = _dinv_block(deg_ref, pl.program_id(0))
    h = jnp.dot(x_ref[...], w_ref[...], preferred_element_type=_f32) + b_ref[...]
    g_ref[...] = dinv * h


def _dense2_body(deg_ref, s_ref, g_ref, w_ref, b_ref, g2_ref):
    dinv = _dinv_block(deg_ref, pl.program_id(0))
    x2 = jnp.maximum(dinv * (s_ref[0] + s_ref[1] + g_ref[...]), 0.0)
    h = jnp.dot(x2, w_ref[...], preferred_element_type=_f32) + b_ref[...]
    g2_ref[...] = dinv * h


def _dense3_body(deg_ref, s_ref, g_ref, o_ref):
    dinv = _dinv_block(deg_ref, pl.program_id(0))
    o_ref[...] = dinv * (s_ref[0] + s_ref[1] + g_ref[...])


_deg_spec = pl.BlockSpec((2, BR, 1), lambda i: (0, i, 0))
_row_spec = pl.BlockSpec((BR, D), lambda i: (i, 0))
_s_spec = pl.BlockSpec((2, BR, D), lambda i: (0, i, 0))
_w_spec = pl.BlockSpec((D, D), lambda i: (0, 0))
_b_spec = pl.BlockSpec((1, D), lambda i: (0, 0))

_full_out = jax.ShapeDtypeStruct((NPAD, D), _f32)

_dense1 = pl.pallas_call(
    _dense1_body, grid=(GRID,),
    in_specs=[_deg_spec, _row_spec, _w_spec, _b_spec],
    out_specs=_row_spec,
    out_shape=_full_out,
)
_dense2 = pl.pallas_call(
    _dense2_body, grid=(GRID,),
    in_specs=[_deg_spec, _s_spec, _row_spec, _w_spec, _b_spec],
    out_specs=_row_spec,
    out_shape=_full_out,
)
_dense3 = pl.pallas_call(
    _dense3_body, grid=(GRID,),
    in_specs=[_deg_spec, _s_spec, _row_spec],
    out_specs=_row_spec,
    out_shape=_full_out,
)


def kernel(x, edge_index_org, W1, b1, W2, b2):
    pad1 = N + jnp.arange(EPAD - E, dtype=jnp.int32) % (NPAD - N)
    pad = jnp.stack([pad1, pad1], axis=0)
    ei = jnp.concatenate([edge_index_org.astype(jnp.int32), pad], axis=1)
    rows2d = ei[0].reshape(EPAD // 128, 128)
    cols2d = ei[1].reshape(EPAD // 128, 128)
    rowsa2d = (rows2d * 2).reshape(EPAD // 256, 256)
    rowsb2d = rowsa2d + 1
    cols2d_w = cols2d.reshape(EPAD // 256, 256)

    x_pad = jnp.concatenate([x, jnp.zeros((NPAD - N, D), _f32)], axis=0)
    ones8 = jnp.ones((128, 8), _f32)
    zeros8 = jnp.zeros((ROWS_PER_TILE, 8), _f32)
    zeros_stage = jnp.zeros((ROWS_PER_TILE, H), _f32)
    b1r = b1.reshape(1, D)
    b2r = b2.reshape(1, D)

    deg8 = _hist(rows2d, ones8, zeros8)
    deg = deg8[:, :, 0:1]

    g1 = _dense1(deg, x_pad, W1, b1r)
    s1 = _gs(g1.reshape(2 * NPAD, H), rowsa2d, rowsb2d, cols2d_w, zeros_stage)
    g2 = _dense2(deg, s1, g1, W2, b2r)
    s2 = _gs(g2.reshape(2 * NPAD, H), rowsa2d, rowsb2d, cols2d_w, zeros_stage)
    out = _dense3(deg, s2, g2)
    return out[:N]


# speedup vs baseline: 1.0773x; 1.0773x over previous
import functools

import jax
import jax.numpy as jnp
from jax import lax
from jax.experimental import pallas as pl
from jax.experimental.pallas import tpu as pltpu
from jax.experimental.pallas import tpu_sc as plsc

N = 10000
D = 128
H = D // 2
E = 320000

NPAD = 10240
EPAD = 327680
CHUNKS = 80
ROWS_PER_TILE = NPAD // 16
BR = 512
GRID = NPAD // BR

_mesh = plsc.VectorSubcoreMesh(core_axis_name="c", subcore_axis_name="s")
_f32 = jnp.float32
_sc_params = pltpu.CompilerParams(use_tc_tiling_on_sc=False)


@functools.partial(
    pl.kernel,
    out_type=jax.ShapeDtypeStruct((2, NPAD, 8), _f32),
    mesh=_mesh,
    scratch_types=[
        pltpu.VMEM((CHUNKS, 128), jnp.int32),
        pltpu.VMEM((128, 8), _f32),
        pltpu.VMEM((ROWS_PER_TILE, 8), _f32),
        pltpu.VMEM_SHARED((NPAD, 8), _f32),
    ],
    compiler_params=_sc_params,
)
def _hist(rows_hbm, ones8_hbm, zeros8_hbm, out_hbm, rowbuf, onesv, stage, acc):
    cid = lax.axis_index("c")
    sid = lax.axis_index("s")
    w = sid * 2 + cid
    pltpu.sync_copy(rows_hbm.at[pl.ds(w * CHUNKS, CHUNKS)], rowbuf)
    pltpu.sync_copy(ones8_hbm, onesv)
    pltpu.sync_copy(zeros8_hbm, stage)
    pltpu.sync_copy(stage, acc.at[pl.ds(sid * ROWS_PER_TILE, ROWS_PER_TILE)])
    plsc.subcore_barrier()

    def body(c, carry):
        pltpu.sync_copy(onesv, acc.at[rowbuf.at[c]], add=True)
        return carry

    lax.fori_loop(0, CHUNKS, body, 0)
    plsc.subcore_barrier()
    pltpu.sync_copy(acc.at[pl.ds(sid * ROWS_PER_TILE, ROWS_PER_TILE)], stage)
    pltpu.sync_copy(stage, out_hbm.at[cid, pl.ds(sid * ROWS_PER_TILE, ROWS_PER_TILE)])


@functools.partial(
    pl.kernel,
    out_type=jax.ShapeDtypeStruct((2, NPAD, D), _f32),
    mesh=_mesh,
    scratch_types=[
        pltpu.VMEM((CHUNKS, 128), jnp.int32),
        pltpu.VMEM((CHUNKS, 128), jnp.int32),
        pltpu.VMEM((128, H), _f32),
        pltpu.VMEM((128, H), _f32),
        pltpu.VMEM((128, H), _f32),
        pltpu.VMEM((128, H), _f32),
        pltpu.VMEM((128, H), _f32),
        pltpu.VMEM((128, H), _f32),
        pltpu.VMEM((128, H), _f32),
        pltpu.VMEM_SHARED((NPAD, H), _f32),
        pltpu.SemaphoreType.DMA,
        pltpu.SemaphoreType.DMA,
        pltpu.SemaphoreType.DMA,
        pltpu.SemaphoreType.DMA,
        pltpu.SemaphoreType.DMA,
        pltpu.SemaphoreType.DMA,
        pltpu.SemaphoreType.DMA,
        pltpu.SemaphoreType.DMA,
        pltpu.SemaphoreType.DMA,
        pltpu.SemaphoreType.DMA,
        pltpu.SemaphoreType.DMA,
        pltpu.SemaphoreType.DMA,
        pltpu.SemaphoreType.DMA,
        pltpu.SemaphoreType.DMA,
    ],
    compiler_params=_sc_params,
)
def _gs(g2_hbm, rowsa_hbm, rowsb_hbm, cols_hbm, zeros_hbm, out_hbm,
        rowbuf, colbuf, gb0, gb1, gb2, gb3, gb4, gb5, gb6, acc,
        gs0, gs1, gs2, gs3, gs4, gs5, gs6, ss0, ss1, ss2, ss3, ss4, ss5, ss6):
    cid = lax.axis_index("c")
    sid = lax.axis_index("s")
    w = sid * 2 + cid
    gbufs = (gb0, gb1, gb2, gb3, gb4, gb5, gb6)
    gsems = (gs0, gs1, gs2, gs3, gs4, gs5, gs6)
    ssems = (ss0, ss1, ss2, ss3, ss4, ss5, ss6)
    pltpu.sync_copy(cols_hbm.at[pl.ds(w * CHUNKS, CHUNKS)], colbuf)

    for rows_hbm, coff in ((rowsa_hbm, 0), (rowsb_hbm, H)):
        pltpu.sync_copy(rows_hbm.at[pl.ds(w * CHUNKS, CHUNKS)], rowbuf)
        pltpu.sync_copy(zeros_hbm, acc.at[pl.ds(sid * ROWS_PER_TILE, ROWS_PER_TILE)])
        plsc.subcore_barrier()

        def _gather_desc(c, b):
            return pltpu.make_async_copy(g2_hbm.at[rowbuf.at[c]], gbufs[b], gsems[b])

        def _scatter_desc(c, b):
            return pltpu.make_async_copy(gbufs[b], acc.at[colbuf.at[c]], ssems[b])

        def _step(c, b):
            _gather_desc(c, b).wait()
            _scatter_desc(c, b).start(add=True)
            b2 = (b + 5) % 7

            @pl.when(c >= 2)
            def _():
                _scatter_desc(c - 2, b2).wait()

            @pl.when(c + 5 < CHUNKS)
            def _():
                _gather_desc(c + 5, b2).start()

        for b0 in range(5):
            _gather_desc(b0, b0).start()

        def body(g, carry):
            for b in range(7):
                _step(g * 7 + b, b)
            return carry

        ngroups = (CHUNKS - 2) // 7
        lax.fori_loop(0, ngroups, body, 0)
        for c in range(ngroups * 7, CHUNKS):
            _step(c, c % 7)
        _scatter_desc(CHUNKS - 2, (CHUNKS - 2) % 7).wait()
        _scatter_desc(CHUNKS - 1, (CHUNKS - 1) % 7).wait()
        plsc.subcore_barrier()
        pltpu.sync_copy(
            acc.at[pl.ds(sid * ROWS_PER_TILE, ROWS_PER_TILE)],
            out_hbm.at[cid, pl.ds(sid * ROWS_PER_TILE, ROWS_PER_TILE), pl.ds(coff, H)])


def _dinv_block(deg_ref, i):
    deg = deg_ref[0] + deg_ref[1] + 1.0
    rows = lax.broadcasted_iota(jnp.int32, (BR, 1), 0) + i * BR
    return jnp.where(rows < N, lax.rsqrt(deg), 0.0)


def _dense1_body(deg_ref, x_ref, w_ref, b_ref, g_ref):
    dinv = _dinv_block(deg_ref, pl.program_id(0))
    h = jnp.dot(x_ref[...], w_ref[...], preferred_element_type=_f32) + b_ref[...]
    g_ref[...] = dinv * h


def _dense2_body(deg_ref, s_ref, g_ref, w_ref, b_ref, g2_ref):
    dinv = _dinv_block(deg_ref, pl.program_id(0))
    x2 = jnp.maximum(dinv * (s_ref[0] + s_ref[1] + g_ref[...]), 0.0)
    h = jnp.dot(x2, w_ref[...], preferred_element_type=_f32) + b_ref[...]
    g2_ref[...] = dinv * h


def _dense3_body(deg_ref, s_ref, g_ref, o_ref):
    dinv = _dinv_block(deg_ref, pl.program_id(0))
    o_ref[...] = dinv * (s_ref[0] + s_ref[1] + g_ref[...])


_deg_spec = pl.BlockSpec((2, BR, 1), lambda i: (0, i, 0))
_row_spec = pl.BlockSpec((BR, D), lambda i: (i, 0))
_s_spec = pl.BlockSpec((2, BR, D), lambda i: (0, i, 0))
_w_spec = pl.BlockSpec((D, D), lambda i: (0, 0))
_b_spec = pl.BlockSpec((1, D), lambda i: (0, 0))

_full_out = jax.ShapeDtypeStruct((NPAD, D), _f32)

_dense1 = pl.pallas_call(
    _dense1_body, grid=(GRID,),
    in_specs=[_deg_spec, _row_spec, _w_spec, _b_spec],
    out_specs=_row_spec,
    out_shape=_full_out,
)
_dense2 = pl.pallas_call(
    _dense2_body, grid=(GRID,),
    in_specs=[_deg_spec, _s_spec, _row_spec, _w_spec, _b_spec],
    out_specs=_row_spec,
    out_shape=_full_out,
)
_dense3 = pl.pallas_call(
    _dense3_body, grid=(GRID,),
    in_specs=[_deg_spec, _s_spec, _row_spec],
    out_specs=_row_spec,
    out_shape=_full_out,
)


def kernel(x, edge_index_org, W1, b1, W2, b2):
    pad1 = N + jnp.arange(EPAD - E, dtype=jnp.int32) % (NPAD - N)
    pad = jnp.stack([pad1, pad1], axis=0)
    ei = jnp.concatenate([edge_index_org.astype(jnp.int32), pad], axis=1)
    rows2d = ei[0].reshape(EPAD // 128, 128)
    cols2d = ei[1].reshape(EPAD // 128, 128)
    rowsa2d = rows2d * 2
    rowsb2d = rowsa2d + 1

    x_pad = jnp.concatenate([x, jnp.zeros((NPAD - N, D), _f32)], axis=0)
    ones8 = jnp.ones((128, 8), _f32)
    zeros8 = jnp.zeros((ROWS_PER_TILE, 8), _f32)
    zeros_stage = jnp.zeros((ROWS_PER_TILE, H), _f32)
    b1r = b1.reshape(1, D)
    b2r = b2.reshape(1, D)

    deg8 = _hist(rows2d, ones8, zeros8)
    deg = deg8[:, :, 0:1]

    g1 = _dense1(deg, x_pad, W1, b1r)
    s1 = _gs(g1.reshape(2 * NPAD, H), rowsa2d, rowsb2d, cols2d, zeros_stage)
    g2 = _dense2(deg, s1, g1, W2, b2r)
    s2 = _gs(g2.reshape(2 * NPAD, H), rowsa2d, rowsb2d, cols2d, zeros_stage)
    out = _dense3(deg, s2, g2)
    return out[:N]
